# baseline (speedup 1.0000x reference)
import functools
import math

import jax
import jax.numpy as jnp
from jax import lax
from jax.experimental import pallas as pl
from jax.experimental.pallas import tpu as pltpu
from jax.experimental.pallas import tpu_sc as plsc


def _make_mbuild(nrows_p, kv, t, ch, fields):
    info = plsc.get_sparse_core_info()
    nc, ns, nl = info.num_cores, info.num_subcores, info.num_lanes
    nw = nc * ns
    rows_pw = nrows_p // nw
    nch = rows_pw // ch
    assert nch >= 2 and nch % 2 == 0
    jt = t // nl
    mesh = plsc.VectorSubcoreMesh(core_axis_name="c", subcore_axis_name="s")

    @functools.partial(
        pl.kernel, mesh=mesh,
        out_type=jax.ShapeDtypeStruct((nrows_p, kv), jnp.int32),
        scratch_types=[
            pltpu.VMEM((ch * fields * jt, nl), jnp.int32),
            pltpu.VMEM((ch * fields * jt, nl), jnp.int32),
            pltpu.VMEM((ch, kv), jnp.int32),
            pltpu.VMEM((ch, kv), jnp.int32),
            pltpu.SemaphoreType.DMA,
            pltpu.SemaphoreType.DMA,
        ],
        compiler_params=pltpu.CompilerParams(needs_layout_passes=False),
    )
    def mbuild(idx_hbm, zeros_hbm, m_hbm, idx_v0, idx_v1, m_v0, m_v1,
               sem0, sem1):
        wid = lax.axis_index("s") * nc + lax.axis_index("c")
        base = wid * rows_pw
        idx_v = (idx_v0, idx_v1)
        m_v = (m_v0, m_v1)
        sem = (sem0, sem1)
        wvecs = [jnp.full((nl,), 1 << (8 * k), dtype=jnp.int32)
                 for k in range(fields)]
        zvec = jnp.zeros((nl,), dtype=jnp.int32)

        def scatter(buf, idxbuf, zero):
            for r in range(ch):
                rvec = jnp.full((nl,), r, dtype=jnp.int32)
                for k in range(fields):
                    for j in range(jt):
                        vals = idxbuf[(r * fields + k) * jt + j]
                        if zero:
                            plsc.store_scatter(buf, [rvec, vals], zvec)
                        else:
                            plsc.addupdate_scatter(buf, [rvec, vals],
                                                   wvecs[k])

        def load_scatter_start(c, b):
            row0 = base + c * ch
            pltpu.sync_copy(
                idx_hbm.at[pl.ds(row0 * fields * jt, ch * fields * jt)],
                idx_v[b])
            scatter(m_v[b], idx_v[b], False)
            pltpu.async_copy(m_v[b], m_hbm.at[pl.ds(row0, ch)], sem[b])

        pltpu.sync_copy(zeros_hbm, m_v0)
        pltpu.sync_copy(zeros_hbm, m_v1)
        for b in range(2):
            load_scatter_start(b, b)

        def pair_body(i, carry):
            for b in range(2):
                c = 2 + i * 2 + b
                row0 = base + c * ch
                pltpu.make_async_copy(
                    m_v[b], m_hbm.at[pl.ds(row0, ch)], sem[b]).wait()
                scatter(m_v[b], idx_v[b], True)
                load_scatter_start(c, b)
            return carry

        lax.fori_loop(0, (nch - 2) // 2, pair_body, 0)

        for b in range(2):
            row0 = base + (nch - 2 + b) * ch
            pltpu.make_async_copy(
                m_v[b], m_hbm.at[pl.ds(row0, ch)], sem[b]).wait()

    return mbuild


def _attn_body(q_ref, k_ref, v_ref, m_ref, o_ref, *, G):
    k = k_ref[0]
    v = v_ref[0]
    mp = m_ref[0]
    pid = pl.program_id(1)
    cnt = lax.shift_right_logical(mp, pid * 8) & 255
    logm = jnp.log(cnt.astype(jnp.float32))
    for g in range(G):
        q = q_ref[0, g]
        s = lax.dot_general(q, k, (((1,), (1,)), ((), ())),
                            preferred_element_type=jnp.float32)
        s = s + logm
        mx = jnp.max(s, axis=1, keepdims=True)
        w = jnp.exp(s - mx)
        denom = jnp.sum(w, axis=1, keepdims=True)
        o = lax.dot_general(w.astype(jnp.bfloat16), v,
                            (((1,), (0,)), ((), ())),
                            preferred_element_type=jnp.float32)
        o_ref[0, g] = o / denom


def kernel(q, k, v, indices):
    B, Hq, S, D = q.shape
    Hkv = k.shape[1]
    KV = k.shape[2]
    G = Hq // Hkv
    T = indices.shape[-1]
    assert B == 1

    F = 4
    S4 = S // F
    CH = 8
    info = plsc.get_sparse_core_info()
    nl = info.num_lanes

    HH = Hkv // 2
    nrows_h = HH * S4

    idx_flat = (indices.reshape(Hkv, F, S4, T).transpose(0, 2, 1, 3)
                .reshape(Hkv, S4 * F * (T // nl), nl).astype(jnp.int32))
    zeros_init = jnp.zeros((CH, KV), jnp.int32)
    mbuild = _make_mbuild(nrows_h, KV, T, CH, F)

    BS = S4
    qr = (q * (1.0 / math.sqrt(D))).reshape(Hkv, G, S, D).astype(jnp.bfloat16)
    kr = k.reshape(Hkv, KV, D).astype(jnp.bfloat16)
    vr = v.reshape(Hkv, KV, D).astype(jnp.bfloat16)

    attn = pl.pallas_call(
        functools.partial(_attn_body, G=G),
        grid=(HH, F),
        in_specs=[
            pl.BlockSpec((1, G, BS, D), lambda h, s: (h, 0, s, 0)),
            pl.BlockSpec((1, KV, D), lambda h, s: (h, 0, 0)),
            pl.BlockSpec((1, KV, D), lambda h, s: (h, 0, 0)),
            pl.BlockSpec((1, S4, KV), lambda h, s: (h, 0, 0)),
        ],
        out_specs=pl.BlockSpec((1, G, BS, D), lambda h, s: (h, 0, s, 0)),
        out_shape=jax.ShapeDtypeStruct((HH, G, S, D), jnp.float32),
        compiler_params=pltpu.CompilerParams(
            dimension_semantics=("parallel", "parallel")),
    )

    outs = []
    for half in range(2):
        sl = slice(half * HH, (half + 1) * HH)
        m_half = mbuild(
            idx_flat[sl].reshape(nrows_h * F * (T // nl), nl), zeros_init)
        outs.append(attn(qr[sl], kr[sl], vr[sl],
                         m_half.reshape(HH, S4, KV)))
    return jnp.concatenate(outs, axis=0).reshape(B, Hq, S, D)


# speedup vs baseline: 1.0418x; 1.0418x over previous
import functools
import math

import jax
import jax.numpy as jnp
from jax import lax
from jax.experimental import pallas as pl
from jax.experimental.pallas import tpu as pltpu
from jax.experimental.pallas import tpu_sc as plsc


def _make_mbuild(nrows_p, s4, seq, kv, t, ch, fields):
    info = plsc.get_sparse_core_info()
    nc, ns, nl = info.num_cores, info.num_subcores, info.num_lanes
    nw = nc * ns
    rows_pw = nrows_p // nw
    nch = rows_pw // ch
    assert nch >= 2 and nch % 2 == 0
    assert s4 % rows_pw == 0
    jt = t // nl
    mesh = plsc.VectorSubcoreMesh(core_axis_name="c", subcore_axis_name="s")

    @functools.partial(
        pl.kernel, mesh=mesh,
        out_type=jax.ShapeDtypeStruct((nrows_p, kv), jnp.int32),
        scratch_types=[
            pltpu.VMEM((ch * fields * jt, nl), jnp.int32),
            pltpu.VMEM((ch * fields * jt, nl), jnp.int32),
            pltpu.VMEM((ch, kv), jnp.int32),
            pltpu.VMEM((ch, kv), jnp.int32),
            pltpu.SemaphoreType.DMA,
            pltpu.SemaphoreType.DMA,
        ],
        compiler_params=pltpu.CompilerParams(needs_layout_passes=False),
    )
    def mbuild(idx_hbm, zeros_hbm, m_hbm, idx_v0, idx_v1, m_v0, m_v1,
               sem0, sem1):
        wid = lax.axis_index("s") * nc + lax.axis_index("c")
        base = wid * rows_pw
        idx_v = (idx_v0, idx_v1)
        m_v = (m_v0, m_v1)
        sem = (sem0, sem1)
        wvecs = [jnp.full((nl,), 1 << (8 * k), dtype=jnp.int32)
                 for k in range(fields)]
        zvec = jnp.zeros((nl,), dtype=jnp.int32)

        def scatter(buf, idxbuf, zero):
            for r in range(ch):
                rvec = jnp.full((nl,), r, dtype=jnp.int32)
                for k in range(fields):
                    for j in range(jt):
                        vals = idxbuf[(k * ch + r) * jt + j]
                        if zero:
                            plsc.store_scatter(buf, [rvec, vals], zvec)
                        else:
                            plsc.addupdate_scatter(buf, [rvec, vals],
                                                   wvecs[k])

        def load_scatter_start(c, b):
            row0 = base + c * ch
            h = row0 // s4
            p0 = row0 - h * s4
            for k in range(fields):
                src = (h * seq + k * s4 + p0) * jt
                pltpu.sync_copy(
                    idx_hbm.at[pl.ds(src, ch * jt)],
                    idx_v[b].at[pl.ds(k * ch * jt, ch * jt)])
            scatter(m_v[b], idx_v[b], False)
            pltpu.async_copy(m_v[b], m_hbm.at[pl.ds(row0, ch)], sem[b])

        pltpu.sync_copy(zeros_hbm, m_v0)
        pltpu.sync_copy(zeros_hbm, m_v1)
        for b in range(2):
            load_scatter_start(b, b)

        def pair_body(i, carry):
            for b in range(2):
                c = 2 + i * 2 + b
                row0 = base + c * ch
                pltpu.make_async_copy(
                    m_v[b], m_hbm.at[pl.ds(row0, ch)], sem[b]).wait()
                scatter(m_v[b], idx_v[b], True)
                load_scatter_start(c, b)
            return carry

        lax.fori_loop(0, (nch - 2) // 2, pair_body, 0)

        for b in range(2):
            row0 = base + (nch - 2 + b) * ch
            pltpu.make_async_copy(
                m_v[b], m_hbm.at[pl.ds(row0, ch)], sem[b]).wait()

    return mbuild


def _attn_body(q_ref, k_ref, v_ref, m_ref, o_ref, kbf_ref, vbf_ref,
               *, G, scale):
    pid = pl.program_id(1)

    @pl.when(pid == 0)
    def _():
        kbf_ref[...] = k_ref[0].astype(jnp.bfloat16)
        vbf_ref[...] = v_ref[0].astype(jnp.bfloat16)

    k = kbf_ref[...]
    v = vbf_ref[...]
    mp = m_ref[0]
    cnt = lax.shift_right_logical(mp, pid * 8) & 255
    logm = jnp.log(cnt.astype(jnp.float32))
    for g in range(G):
        q = (q_ref[0, g] * scale).astype(jnp.bfloat16)
        s = lax.dot_general(q, k, (((1,), (1,)), ((), ())),
                            preferred_element_type=jnp.float32)
        s = s + logm
        mx = jnp.max(s, axis=1, keepdims=True)
        w = jnp.exp(s - mx)
        denom = jnp.sum(w, axis=1, keepdims=True)
        o = lax.dot_general(w.astype(jnp.bfloat16), v,
                            (((1,), (0,)), ((), ())),
                            preferred_element_type=jnp.float32)
        o_ref[0, g] = o / denom


def kernel(q, k, v, indices):
    B, Hq, S, D = q.shape
    Hkv = k.shape[1]
    KV = k.shape[2]
    G = Hq // Hkv
    T = indices.shape[-1]
    assert B == 1

    F = 4
    S4 = S // F
    nrows_p = Hkv * S4
    CH = 8
    info = plsc.get_sparse_core_info()
    nl = info.num_lanes

    idx_flat = indices.reshape(Hkv * S * (T // nl), nl).astype(jnp.int32)
    zeros_init = jnp.zeros((CH, KV), jnp.int32)
    m_packed = _make_mbuild(nrows_p, S4, S, KV, T, CH, F)(idx_flat,
                                                          zeros_init)
    m_packed = m_packed.reshape(Hkv, S4, KV)

    BS = S4
    qr = q.reshape(Hkv, G, S, D)
    kr = k.reshape(Hkv, KV, D)
    vr = v.reshape(Hkv, KV, D)

    out = pl.pallas_call(
        functools.partial(_attn_body, G=G, scale=1.0 / math.sqrt(D)),
        grid=(Hkv, F),
        in_specs=[
            pl.BlockSpec((1, G, BS, D), lambda h, s: (h, 0, s, 0)),
            pl.BlockSpec((1, KV, D), lambda h, s: (h, 0, 0)),
            pl.BlockSpec((1, KV, D), lambda h, s: (h, 0, 0)),
            pl.BlockSpec((1, S4, KV), lambda h, s: (h, 0, 0)),
        ],
        out_specs=pl.BlockSpec((1, G, BS, D), lambda h, s: (h, 0, s, 0)),
        out_shape=jax.ShapeDtypeStruct((Hkv, G, S, D), jnp.float32),
        scratch_shapes=[
            pltpu.VMEM((KV, D), jnp.bfloat16),
            pltpu.VMEM((KV, D), jnp.bfloat16),
        ],
        compiler_params=pltpu.CompilerParams(
            dimension_semantics=("parallel", "arbitrary")),
    )(qr, kr, vr, m_packed)
    return out.reshape(B, Hq, S, D)


# speedup vs baseline: 1.1219x; 1.0769x over previous
import functools
import math

import jax
import jax.numpy as jnp
from jax import lax
from jax.experimental import pallas as pl
from jax.experimental.pallas import tpu as pltpu
from jax.experimental.pallas import tpu_sc as plsc


def _make_mbuild(nrows_p, s4, seq, kv, t, ch, fields):
    info = plsc.get_sparse_core_info()
    nc, ns, nl = info.num_cores, info.num_subcores, info.num_lanes
    nw = nc * ns
    rows_pw = nrows_p // nw
    nch = rows_pw // ch
    assert nch >= 2 and nch % 2 == 0
    assert s4 % rows_pw == 0
    jt = t // nl
    mesh = plsc.VectorSubcoreMesh(core_axis_name="c", subcore_axis_name="s")

    @functools.partial(
        pl.kernel, mesh=mesh,
        out_type=jax.ShapeDtypeStruct((nrows_p, kv), jnp.int32),
        scratch_types=[
            pltpu.VMEM((ch * fields * jt, nl), jnp.int32),
            pltpu.VMEM((ch * fields * jt, nl), jnp.int32),
            pltpu.VMEM((ch, kv), jnp.int32),
            pltpu.VMEM((ch, kv), jnp.int32),
            pltpu.SemaphoreType.DMA,
            pltpu.SemaphoreType.DMA,
            pltpu.SemaphoreType.DMA,
        ],
        compiler_params=pltpu.CompilerParams(needs_layout_passes=False),
    )
    def mbuild(idx_hbm, zeros_hbm, m_hbm, idx_v0, idx_v1, m_v0, m_v1,
               sem0, sem1, isem):
        wid = lax.axis_index("s") * nc + lax.axis_index("c")
        base = wid * rows_pw
        idx_v = (idx_v0, idx_v1)
        m_v = (m_v0, m_v1)
        sem = (sem0, sem1)
        wvecs = [jnp.full((nl,), 1 << (8 * k), dtype=jnp.int32)
                 for k in range(fields)]
        zvec = jnp.zeros((nl,), dtype=jnp.int32)

        def scatter(buf, idxbuf, zero):
            for r in range(ch):
                rvec = jnp.full((nl,), r, dtype=jnp.int32)
                for k in range(fields):
                    for j in range(jt):
                        vals = idxbuf[(k * ch + r) * jt + j]
                        if zero:
                            plsc.store_scatter(buf, [rvec, vals], zvec)
                        else:
                            plsc.addupdate_scatter(buf, [rvec, vals],
                                                   wvecs[k])

        def load_scatter_start(c, b):
            row0 = base + c * ch
            h = row0 // s4
            p0 = row0 - h * s4
            copies = []
            for k in range(fields):
                src = (h * seq + k * s4 + p0) * jt
                cp = pltpu.make_async_copy(
                    idx_hbm.at[pl.ds(src, ch * jt)],
                    idx_v[b].at[pl.ds(k * ch * jt, ch * jt)], isem)
                cp.start()
                copies.append(cp)
            for cp in copies:
                cp.wait()
            scatter(m_v[b], idx_v[b], False)
            pltpu.async_copy(m_v[b], m_hbm.at[pl.ds(row0, ch)], sem[b])

        pltpu.sync_copy(zeros_hbm, m_v0)
        pltpu.sync_copy(zeros_hbm, m_v1)
        for b in range(2):
            load_scatter_start(b, b)

        def pair_body(i, carry):
            for b in range(2):
                c = 2 + i * 2 + b
                row0 = base + c * ch
                pltpu.make_async_copy(
                    m_v[b], m_hbm.at[pl.ds(row0, ch)], sem[b]).wait()
                scatter(m_v[b], idx_v[b], True)
                load_scatter_start(c, b)
            return carry

        lax.fori_loop(0, (nch - 2) // 2, pair_body, 0)

        for b in range(2):
            row0 = base + (nch - 2 + b) * ch
            pltpu.make_async_copy(
                m_v[b], m_hbm.at[pl.ds(row0, ch)], sem[b]).wait()

    return mbuild


def _attn_body(q_ref, k_ref, v_ref, m_ref, o_ref, kbf_ref, vbf_ref,
               *, G, scale):
    pid = pl.program_id(1)

    @pl.when(pid == 0)
    def _():
        kbf_ref[...] = k_ref[0].astype(jnp.bfloat16)
        vbf_ref[...] = v_ref[0].astype(jnp.bfloat16)

    k = kbf_ref[...]
    v = vbf_ref[...]
    mp = m_ref[0]
    cnt = lax.shift_right_logical(mp, pid * 8) & 255
    logm = jnp.log(cnt.astype(jnp.float32))
    for g in range(G):
        q = (q_ref[0, g] * scale).astype(jnp.bfloat16)
        s = lax.dot_general(q, k, (((1,), (1,)), ((), ())),
                            preferred_element_type=jnp.float32)
        s = s + logm
        mx = jnp.max(s, axis=1, keepdims=True)
        w = jnp.exp(s - mx)
        denom = jnp.sum(w, axis=1, keepdims=True)
        o = lax.dot_general(w.astype(jnp.bfloat16), v,
                            (((1,), (0,)), ((), ())),
                            preferred_element_type=jnp.float32)
        o_ref[0, g] = o / denom


def kernel(q, k, v, indices):
    B, Hq, S, D = q.shape
    Hkv = k.shape[1]
    KV = k.shape[2]
    G = Hq // Hkv
    T = indices.shape[-1]
    assert B == 1

    F = 4
    S4 = S // F
    nrows_p = Hkv * S4
    CH = 8
    info = plsc.get_sparse_core_info()
    nl = info.num_lanes

    idx_flat = indices.reshape(Hkv * S * (T // nl), nl).astype(jnp.int32)
    zeros_init = jnp.zeros((CH, KV), jnp.int32)
    m_packed = _make_mbuild(nrows_p, S4, S, KV, T, CH, F)(idx_flat,
                                                          zeros_init)
    m_packed = m_packed.reshape(Hkv, S4, KV)

    BS = S4
    qr = q.reshape(Hkv, G, S, D)
    kr = k.reshape(Hkv, KV, D)
    vr = v.reshape(Hkv, KV, D)

    out = pl.pallas_call(
        functools.partial(_attn_body, G=G, scale=1.0 / math.sqrt(D)),
        grid=(Hkv, F),
        in_specs=[
            pl.BlockSpec((1, G, BS, D), lambda h, s: (h, 0, s, 0)),
            pl.BlockSpec((1, KV, D), lambda h, s: (h, 0, 0)),
            pl.BlockSpec((1, KV, D), lambda h, s: (h, 0, 0)),
            pl.BlockSpec((1, S4, KV), lambda h, s: (h, 0, 0)),
        ],
        out_specs=pl.BlockSpec((1, G, BS, D), lambda h, s: (h, 0, s, 0)),
        out_shape=jax.ShapeDtypeStruct((Hkv, G, S, D), jnp.float32),
        scratch_shapes=[
            pltpu.VMEM((KV, D), jnp.bfloat16),
            pltpu.VMEM((KV, D), jnp.bfloat16),
        ],
        compiler_params=pltpu.CompilerParams(
            dimension_semantics=("parallel", "arbitrary")),
    )(qr, kr, vr, m_packed)
    return out.reshape(B, Hq, S, D)
